# baseline (speedup 1.0000x reference)
import functools

import jax
import jax.numpy as jnp
from jax import lax
from jax.experimental import pallas as pl
from jax.experimental.pallas import tpu as pltpu
from jax.experimental.pallas import tpu_sc as plsc


def _make_copy_kernel(n_rows, n_feat, dtype):
    info = plsc.get_sparse_core_info()
    nc, ns = info.num_cores, info.num_subcores
    nw = nc * ns
    rows_per = n_rows // nw
    chunk = min(128, rows_per)
    n_chunks = rows_per // chunk
    mesh = plsc.VectorSubcoreMesh(core_axis_name="c", subcore_axis_name="s")

    @functools.partial(
        pl.kernel,
        mesh=mesh,
        out_type=jax.ShapeDtypeStruct((n_rows, n_feat), dtype),
        scratch_types=(
            [pltpu.VMEM((chunk, n_feat), dtype) for _ in range(n_chunks)]
            + [pltpu.SemaphoreType.DMA for _ in range(2 * n_chunks)]
        ),
    )
    def body(x_hbm, out_hbm, *scratch):
        bufs = scratch[:n_chunks]
        gsems = scratch[n_chunks : 2 * n_chunks]
        ssems = scratch[2 * n_chunks :]
        wid = lax.axis_index("s") * nc + lax.axis_index("c")
        base = wid * rows_per

        for i in range(n_chunks):
            pltpu.make_async_copy(
                x_hbm.at[pl.ds(base + i * chunk, chunk)], bufs[i], gsems[i]
            ).start()
        for i in range(n_chunks):
            pltpu.make_async_copy(
                x_hbm.at[pl.ds(base + i * chunk, chunk)], bufs[i], gsems[i]
            ).wait()
            pltpu.make_async_copy(
                bufs[i], out_hbm.at[pl.ds(base + i * chunk, chunk)], ssems[i]
            ).start()
        for i in range(n_chunks):
            pltpu.make_async_copy(
                bufs[i], out_hbm.at[pl.ds(base + i * chunk, chunk)], ssems[i]
            ).wait()

    return body


def kernel(x, queue):
    n_out = min(x.shape[0], queue.shape[0])
    return _make_copy_kernel(n_out, x.shape[1], x.dtype)(x[:n_out])
